# baseline (speedup 1.0000x reference)
import functools

import jax
import jax.numpy as jnp
from jax import lax
from jax.experimental import pallas as pl
from jax.experimental.pallas import tpu as pltpu
from jax.experimental.pallas import tpu_sc as plsc

_B, _N_MEL, _N_TEXT, _D = 8, 1024, 256, 256
_CH = 64
_NCH = _N_MEL // _CH
_NSUB = _N_TEXT // 16
_INF = float("inf")


def _bmm_kernel(mel_ref, text_ref, out_ref):
    out_ref[0] = jax.lax.dot_general(
        mel_ref[0], text_ref[0], (((1,), (1,)), ((), ())),
        preferred_element_type=jnp.float32)


def _fwd_row(lp_ref, prev_ref, m_ref, r):
    lane = lax.iota(jnp.int32, 16)
    for k in range(_NSUB - 1, -1, -1):
        pv = prev_ref[pl.ds(k * 16, 16)]
        if k > 0:
            sh = plsc.load_gather(prev_ref, [lane + (k * 16 - 1)])
        else:
            g = plsc.load_gather(prev_ref, [lax.max(lane - 1, 0)])
            sh = jnp.where(lane == 0, jnp.float32(_INF), g)
        m_ref[r, pl.ds(k * 16, 16)] = jnp.where(sh <= pv, 1.0, 0.0)
        cur = jnp.minimum(pv, sh) - lp_ref[r, pl.ds(k * 16, 16)]
        prev_ref[pl.ds(k * 16, 16)] = cur


def _sc_dp_body(lp_hbm, out_hbm, lp_v, prev_v, m_v, jcol_v, out_v, m_sp):
    c = lax.axis_index("c")
    s = lax.axis_index("s")
    b = s * 2 + c

    @pl.when(b < _B)
    def _run():
        lane = lax.iota(jnp.int32, 16)
        ones = jnp.ones((16,), jnp.float32)
        zeros = jnp.zeros((16,), jnp.float32)

        pltpu.sync_copy(lp_hbm.at[b, pl.ds(0, _CH)], lp_v)
        for k in range(_NSUB):
            row0 = jnp.where((lane == 0) & (k == 0),
                             -lp_v[0, pl.ds(k * 16, 16)], jnp.float32(_INF))
            prev_v[pl.ds(k * 16, 16)] = row0

        def _row_c0(r, _):
            _fwd_row(lp_v, prev_v, m_v, r)
            return 0

        lax.fori_loop(1, _CH, _row_c0, 0)
        pltpu.sync_copy(m_v, m_sp.at[s, pl.ds(0, _CH)])

        def _chunk(ci, _):
            pltpu.sync_copy(lp_hbm.at[b, pl.ds(ci * _CH, _CH)], lp_v)

            def _row(r, _2):
                _fwd_row(lp_v, prev_v, m_v, r)
                return 0

            lax.fori_loop(0, _CH, _row, 0)
            pltpu.sync_copy(m_v, m_sp.at[s, pl.ds(ci * _CH, _CH)])
            return 0

        lax.fori_loop(1, _NCH, _chunk, 0)

        def _zrow(r, _):
            for k in range(_NSUB):
                out_v[r, pl.ds(k * 16, 16)] = zeros
            return 0

        lax.fori_loop(0, _CH, _zrow, 0)

        jvec0 = jnp.where(lane == 15, jnp.int32(_N_TEXT - 1), 0)

        def _bchunk(t, carry):
            ci = _NCH - 1 - t
            pltpu.sync_copy(m_sp.at[s, pl.ds(ci * _CH, _CH)], m_v)

            def _bstep(u, carry):
                jj, jvec = carry
                x = ci * _CH + _CH - 1 - u
                jc = pl.multiple_of((jj // 16) * 16, 16)
                vec = m_v[x - ci * _CH, pl.ds(jc, 16)]
                sel = jnp.where(lane == (jj - jc), vec, 0.0)
                mv = lax.reduce_max(sel, (0,)).astype(jnp.int32)
                jn = jj - mv
                jvec = jnp.where(lane == ((x - 1) % 16), jn, jvec)

                @pl.when(((x - 1) % 16 == 0) & (x > 0))
                def _store():
                    jcol_v[pl.ds(pl.multiple_of(x - 1, 16), 16)] = jvec

                return jn, jvec

            return lax.fori_loop(0, _CH, _bstep, carry)

        lax.fori_loop(0, _NCH, _bchunk, (jnp.int32(_N_TEXT - 1), jvec0))

        def _ochunk(co, _):
            for q in range(_CH // 16):
                rows = lane + q * 16
                cols = jcol_v[pl.ds(pl.multiple_of(co * _CH + q * 16, 16), 16)]
                plsc.store_scatter(out_v, [rows, cols], ones)
            pltpu.sync_copy(out_v, out_hbm.at[b, pl.ds(co * _CH, _CH)])
            for q in range(_CH // 16):
                rows = lane + q * 16
                cols = jcol_v[pl.ds(pl.multiple_of(co * _CH + q * 16, 16), 16)]
                plsc.store_scatter(out_v, [rows, cols], zeros)
            return 0

        lax.fori_loop(0, _NCH, _ochunk, 0)


@functools.partial(
    pl.kernel,
    out_type=jax.ShapeDtypeStruct((_B, _N_MEL, _N_TEXT), jnp.float32),
    mesh=plsc.VectorSubcoreMesh(core_axis_name="c", subcore_axis_name="s",
                                num_cores=2, num_subcores=16),
    compiler_params=pltpu.CompilerParams(needs_layout_passes=False),
    scratch_types=[
        pltpu.VMEM((_CH, _N_TEXT), jnp.float32),
        pltpu.VMEM((_N_TEXT,), jnp.float32),
        pltpu.VMEM((_CH, _N_TEXT), jnp.float32),
        pltpu.VMEM((_N_MEL,), jnp.int32),
        pltpu.VMEM((_CH, _N_TEXT), jnp.float32),
        pltpu.VMEM_SHARED((4, _N_MEL, _N_TEXT), jnp.float32),
    ],
)
def _sc_dp(lp_hbm, out_hbm, lp_v, prev_v, m_v, jcol_v, out_v, m_sp):
    _sc_dp_body(lp_hbm, out_hbm, lp_v, prev_v, m_v, jcol_v, out_v, m_sp)


def kernel(text_emb, mel_emb):
    b, n_mel, d = mel_emb.shape
    n_text = text_emb.shape[1]
    lp = pl.pallas_call(
        _bmm_kernel,
        grid=(b,),
        in_specs=[
            pl.BlockSpec((1, n_mel, d), lambda i: (i, 0, 0)),
            pl.BlockSpec((1, n_text, d), lambda i: (i, 0, 0)),
        ],
        out_specs=pl.BlockSpec((1, n_mel, n_text), lambda i: (i, 0, 0)),
        out_shape=jax.ShapeDtypeStruct((b, n_mel, n_text), jnp.float32),
    )(mel_emb, text_emb)
    return _sc_dp(lp)


# speedup vs baseline: 1.7307x; 1.7307x over previous
import functools

import jax
import jax.numpy as jnp
from jax import lax
from jax.experimental import pallas as pl
from jax.experimental.pallas import tpu as pltpu
from jax.experimental.pallas import tpu_sc as plsc

_B, _N_MEL, _N_TEXT, _D = 8, 1024, 256, 256
_CH = 64
_NCH = _N_MEL // _CH
_NSUB = _N_TEXT // 16
_INF = float("inf")


def _bmm_kernel(mel_ref, text_ref, out_ref):
    out_ref[0] = jax.lax.dot_general(
        mel_ref[0], text_ref[0], (((1,), (1,)), ((), ())),
        preferred_element_type=jnp.float32)


def _fwd_row(lp_ref, m_ref, r, prevs):
    lane = lax.iota(jnp.int32, 16)
    idx_rot = (lane + 15) % 16
    lane0 = lane == 0
    new = []
    rot_prev = None
    for k in range(_NSUB):
        pv = prevs[k]
        rot = jnp.take_along_axis(pv, idx_rot, axis=0)
        if k == 0:
            sh = jnp.where(lane0, jnp.float32(_INF), rot)
        else:
            sh = jnp.where(lane0, rot_prev, rot)
        rot_prev = rot
        m_ref[r, pl.ds(k * 16, 16)] = jnp.where(sh <= pv, 1.0, 0.0)
        new.append(jnp.minimum(pv, sh) - lp_ref[r, pl.ds(k * 16, 16)])
    return tuple(new)


def _sc_dp_body(lp_hbm, out_hbm, lp_v, m_v, jcol_v, out_v, m_sp):
    c = lax.axis_index("c")
    s = lax.axis_index("s")
    b = s * 2 + c

    @pl.when(b < _B)
    def _run():
        lane = lax.iota(jnp.int32, 16)
        ones = jnp.ones((16,), jnp.float32)
        zeros = jnp.zeros((16,), jnp.float32)

        pltpu.sync_copy(lp_hbm.at[b, pl.ds(0, _CH)], lp_v)
        prevs = tuple(
            jnp.where((lane == 0) & (k == 0),
                      -lp_v[0, pl.ds(k * 16, 16)], jnp.float32(_INF))
            for k in range(_NSUB))

        def _row_c0(r, prevs):
            return _fwd_row(lp_v, m_v, r, prevs)

        prevs = lax.fori_loop(1, _CH, _row_c0, prevs)
        pltpu.sync_copy(m_v, m_sp.at[s, pl.ds(0, _CH)])

        def _chunk(ci, prevs):
            pltpu.sync_copy(lp_hbm.at[b, pl.ds(ci * _CH, _CH)], lp_v)

            def _row(r, prevs):
                return _fwd_row(lp_v, m_v, r, prevs)

            prevs = lax.fori_loop(0, _CH, _row, prevs)
            pltpu.sync_copy(m_v, m_sp.at[s, pl.ds(ci * _CH, _CH)])
            return prevs

        lax.fori_loop(1, _NCH, _chunk, prevs)

        def _zrow(r, _):
            for k in range(_NSUB):
                out_v[r, pl.ds(k * 16, 16)] = zeros
            return 0

        lax.fori_loop(0, _CH, _zrow, 0)

        jvec0 = jnp.where(lane == 15, jnp.int32(_N_TEXT - 1), 0)

        def _bchunk(t, carry):
            ci = _NCH - 1 - t
            pltpu.sync_copy(m_sp.at[s, pl.ds(ci * _CH, _CH)], m_v)

            def _bstep(u, carry):
                jj, jvec = carry
                x = ci * _CH + _CH - 1 - u
                row = jnp.full((16,), x - ci * _CH, jnp.int32)
                mv = plsc.load_gather(m_v, [row, jj]).astype(jnp.int32)
                jn = jj - mv
                jvec = jnp.where(lane == ((x - 1) % 16), jn, jvec)

                @pl.when(((x - 1) % 16 == 0) & (x > 0))
                def _store():
                    jcol_v[pl.ds(pl.multiple_of(x - 1, 16), 16)] = jvec

                return jn, jvec

            return lax.fori_loop(0, _CH, _bstep, carry)

        jj0 = jnp.full((16,), _N_TEXT - 1, jnp.int32)
        lax.fori_loop(0, _NCH, _bchunk, (jj0, jvec0))

        def _ochunk(co, _):
            for q in range(_CH // 16):
                rows = lane + q * 16
                cols = jcol_v[pl.ds(pl.multiple_of(co * _CH + q * 16, 16), 16)]
                plsc.store_scatter(out_v, [rows, cols], ones)
            pltpu.sync_copy(out_v, out_hbm.at[b, pl.ds(co * _CH, _CH)])
            for q in range(_CH // 16):
                rows = lane + q * 16
                cols = jcol_v[pl.ds(pl.multiple_of(co * _CH + q * 16, 16), 16)]
                plsc.store_scatter(out_v, [rows, cols], zeros)
            return 0

        lax.fori_loop(0, _NCH, _ochunk, 0)


@functools.partial(
    pl.kernel,
    out_type=jax.ShapeDtypeStruct((_B, _N_MEL, _N_TEXT), jnp.float32),
    mesh=plsc.VectorSubcoreMesh(core_axis_name="c", subcore_axis_name="s",
                                num_cores=2, num_subcores=16),
    compiler_params=pltpu.CompilerParams(needs_layout_passes=False),
    scratch_types=[
        pltpu.VMEM((_CH, _N_TEXT), jnp.float32),
        pltpu.VMEM((_CH, _N_TEXT), jnp.float32),
        pltpu.VMEM((_N_MEL,), jnp.int32),
        pltpu.VMEM((_CH, _N_TEXT), jnp.float32),
        pltpu.VMEM_SHARED((4, _N_MEL, _N_TEXT), jnp.float32),
    ],
)
def _sc_dp(lp_hbm, out_hbm, lp_v, m_v, jcol_v, out_v, m_sp):
    _sc_dp_body(lp_hbm, out_hbm, lp_v, m_v, jcol_v, out_v, m_sp)


def kernel(text_emb, mel_emb):
    b, n_mel, d = mel_emb.shape
    n_text = text_emb.shape[1]
    lp = pl.pallas_call(
        _bmm_kernel,
        grid=(b,),
        in_specs=[
            pl.BlockSpec((1, n_mel, d), lambda i: (i, 0, 0)),
            pl.BlockSpec((1, n_text, d), lambda i: (i, 0, 0)),
        ],
        out_specs=pl.BlockSpec((1, n_mel, n_text), lambda i: (i, 0, 0)),
        out_shape=jax.ShapeDtypeStruct((b, n_mel, n_text), jnp.float32),
    )(mel_emb, text_emb)
    return _sc_dp(lp)


# speedup vs baseline: 1.7988x; 1.0393x over previous
import functools

import jax
import jax.numpy as jnp
from jax import lax
from jax.experimental import pallas as pl
from jax.experimental.pallas import tpu as pltpu
from jax.experimental.pallas import tpu_sc as plsc

_B, _N_MEL, _N_TEXT, _D = 8, 1024, 256, 256
_CH = 64
_NCH = _N_MEL // _CH
_NSUB = _N_TEXT // 16
_INF = float("inf")


def _bmm_kernel(mel_ref, text_ref, out_ref):
    out_ref[0] = jax.lax.dot_general(
        mel_ref[0], text_ref[0], (((1,), (1,)), ((), ())),
        preferred_element_type=jnp.float32)


def _fwd_row(lp_ref, m_ref, r, prevs):
    lane = lax.iota(jnp.int32, 16)
    idx_rot = (lane + 15) % 16
    lane0 = lane == 0
    new = []
    rot_prev = None
    for k in range(_NSUB):
        pv = prevs[k]
        rot = jnp.take_along_axis(pv, idx_rot, axis=0)
        if k == 0:
            sh = jnp.where(lane0, jnp.float32(_INF), rot)
        else:
            sh = jnp.where(lane0, rot_prev, rot)
        rot_prev = rot
        m_ref[r, pl.ds(k * 16, 16)] = jnp.where(sh <= pv, 1.0, 0.0)
        new.append(jnp.minimum(pv, sh) - lp_ref[r, pl.ds(k * 16, 16)])
    return tuple(new)


def _sc_dp_body(lp_hbm, out_hbm, lp_v, m_v, jcol_v, out_v, m_sp):
    c = lax.axis_index("c")
    s = lax.axis_index("s")
    b = s * 2 + c

    @pl.when(b < _B)
    def _run():
        lane = lax.iota(jnp.int32, 16)
        ones = jnp.ones((16,), jnp.float32)
        zeros = jnp.zeros((16,), jnp.float32)

        pltpu.sync_copy(lp_hbm.at[b, pl.ds(0, _CH)], lp_v)
        prevs = tuple(
            jnp.where((lane == 0) & (k == 0),
                      -lp_v[0, pl.ds(k * 16, 16)], jnp.float32(_INF))
            for k in range(_NSUB))

        def _row_c0(r, prevs):
            return _fwd_row(lp_v, m_v, r, prevs)

        prevs = lax.fori_loop(1, _CH, _row_c0, prevs, unroll=3)
        pltpu.sync_copy(m_v, m_sp.at[s, pl.ds(0, _CH)])

        def _chunk(ci, prevs):
            pltpu.sync_copy(lp_hbm.at[b, pl.ds(ci * _CH, _CH)], lp_v)

            def _row(r, prevs):
                return _fwd_row(lp_v, m_v, r, prevs)

            prevs = lax.fori_loop(0, _CH, _row, prevs, unroll=4)
            pltpu.sync_copy(m_v, m_sp.at[s, pl.ds(ci * _CH, _CH)])
            return prevs

        lax.fori_loop(1, _NCH, _chunk, prevs)

        def _zrow(r, _):
            for k in range(_NSUB):
                out_v[r, pl.ds(k * 16, 16)] = zeros
            return 0

        lax.fori_loop(0, _CH, _zrow, 0)

        jvec0 = jnp.where(lane == 15, jnp.int32(_N_TEXT - 1), 0)

        def _bchunk(t, carry):
            ci = _NCH - 1 - t
            pltpu.sync_copy(m_sp.at[s, pl.ds(ci * _CH, _CH)], m_v)

            def _bstep(u, carry):
                jj, jvec = carry
                x = ci * _CH + _CH - 1 - u
                row = jnp.full((16,), x - ci * _CH, jnp.int32)
                mv = plsc.load_gather(m_v, [row, jj]).astype(jnp.int32)
                jn = jj - mv
                jvec = jnp.where(lane == ((x - 1) % 16), jn, jvec)

                @pl.when(((x - 1) % 16 == 0) & (x > 0))
                def _store():
                    jcol_v[pl.ds(pl.multiple_of(x - 1, 16), 16)] = jvec

                return jn, jvec

            return lax.fori_loop(0, _CH, _bstep, carry, unroll=8)

        jj0 = jnp.full((16,), _N_TEXT - 1, jnp.int32)
        lax.fori_loop(0, _NCH, _bchunk, (jj0, jvec0))

        def _ochunk(co, _):
            for q in range(_CH // 16):
                rows = lane + q * 16
                cols = jcol_v[pl.ds(pl.multiple_of(co * _CH + q * 16, 16), 16)]
                plsc.store_scatter(out_v, [rows, cols], ones)
            pltpu.sync_copy(out_v, out_hbm.at[b, pl.ds(co * _CH, _CH)])
            for q in range(_CH // 16):
                rows = lane + q * 16
                cols = jcol_v[pl.ds(pl.multiple_of(co * _CH + q * 16, 16), 16)]
                plsc.store_scatter(out_v, [rows, cols], zeros)
            return 0

        lax.fori_loop(0, _NCH, _ochunk, 0)


@functools.partial(
    pl.kernel,
    out_type=jax.ShapeDtypeStruct((_B, _N_MEL, _N_TEXT), jnp.float32),
    mesh=plsc.VectorSubcoreMesh(core_axis_name="c", subcore_axis_name="s",
                                num_cores=2, num_subcores=16),
    compiler_params=pltpu.CompilerParams(needs_layout_passes=False),
    scratch_types=[
        pltpu.VMEM((_CH, _N_TEXT), jnp.float32),
        pltpu.VMEM((_CH, _N_TEXT), jnp.float32),
        pltpu.VMEM((_N_MEL,), jnp.int32),
        pltpu.VMEM((_CH, _N_TEXT), jnp.float32),
        pltpu.VMEM_SHARED((4, _N_MEL, _N_TEXT), jnp.float32),
    ],
)
def _sc_dp(lp_hbm, out_hbm, lp_v, m_v, jcol_v, out_v, m_sp):
    _sc_dp_body(lp_hbm, out_hbm, lp_v, m_v, jcol_v, out_v, m_sp)


def kernel(text_emb, mel_emb):
    b, n_mel, d = mel_emb.shape
    n_text = text_emb.shape[1]
    lp = pl.pallas_call(
        _bmm_kernel,
        grid=(b,),
        in_specs=[
            pl.BlockSpec((1, n_mel, d), lambda i: (i, 0, 0)),
            pl.BlockSpec((1, n_text, d), lambda i: (i, 0, 0)),
        ],
        out_specs=pl.BlockSpec((1, n_mel, n_text), lambda i: (i, 0, 0)),
        out_shape=jax.ShapeDtypeStruct((b, n_mel, n_text), jnp.float32),
    )(mel_emb, text_emb)
    return _sc_dp(lp)


# speedup vs baseline: 2.1258x; 1.1818x over previous
import functools

import jax
import jax.numpy as jnp
from jax import lax
from jax.experimental import pallas as pl
from jax.experimental.pallas import tpu as pltpu
from jax.experimental.pallas import tpu_sc as plsc

_B, _N_MEL, _N_TEXT, _D = 8, 1024, 256, 256
_CH = 64
_OCH = 32
_NCH = _N_MEL // _CH
_NSUB = _N_TEXT // 16
_INF = float("inf")


def _bmm_kernel(mel_ref, text_ref, out_ref):
    out_ref[0] = jax.lax.dot_general(
        mel_ref[0], text_ref[0], (((1,), (1,)), ((), ())),
        preferred_element_type=jnp.float32)


def _fwd_row(lp_ref, m_ref, r, prevs):
    lane = lax.iota(jnp.int32, 16)
    idx_rot = (lane + 15) % 16
    lane0 = lane == 0
    new = []
    rot_prev = None
    for k in range(_NSUB):
        pv = prevs[k]
        rot = jnp.take_along_axis(pv, idx_rot, axis=0)
        if k == 0:
            sh = jnp.where(lane0, jnp.float32(_INF), rot)
        else:
            sh = jnp.where(lane0, rot_prev, rot)
        rot_prev = rot
        m_ref[r, pl.ds(k * 16, 16)] = jnp.where(sh <= pv, 1.0, 0.0)
        new.append(jnp.minimum(pv, sh) - lp_ref[r, pl.ds(k * 16, 16)])
    return tuple(new)


def _sc_dp_body(lp_hbm, out_hbm, lp_v, m_v, jcol_v, out_v, m_sp, sems):
    c = lax.axis_index("c")
    s = lax.axis_index("s")
    b = s * 2 + c

    @pl.when(b < _B)
    def _run():
        lane = lax.iota(jnp.int32, 16)
        ones = jnp.ones((16,), jnp.float32)
        zeros = jnp.zeros((16,), jnp.float32)

        cp0 = pltpu.async_copy(lp_hbm.at[b, pl.ds(0, _CH)], lp_v.at[0],
                               sems.at[0])
        pltpu.async_copy(lp_hbm.at[b, pl.ds(_CH, _CH)], lp_v.at[1], sems.at[1])
        cp0.wait()
        prevs = tuple(
            jnp.where((lane == 0) & (k == 0),
                      -lp_v[0, 0, pl.ds(k * 16, 16)], jnp.float32(_INF))
            for k in range(_NSUB))

        def _row_c0(r, prevs):
            return _fwd_row(lp_v.at[0], m_v, r, prevs)

        prevs = lax.fori_loop(1, _CH, _row_c0, prevs, unroll=3)
        pltpu.sync_copy(m_v, m_sp.at[s, pl.ds(0, _CH)])

        def _chunk(ci, prevs):
            p = lax.rem(ci, 2)
            pltpu.make_async_copy(lp_hbm.at[b, pl.ds(ci * _CH, _CH)],
                                  lp_v.at[p], sems.at[p]).wait()

            @pl.when(ci < _NCH - 1)
            def _prefetch():
                pltpu.async_copy(lp_hbm.at[b, pl.ds((ci + 1) * _CH, _CH)],
                                 lp_v.at[1 - p], sems.at[1 - p])

            def _row(r, prevs):
                return _fwd_row(lp_v.at[p], m_v, r, prevs)

            prevs = lax.fori_loop(0, _CH, _row, prevs, unroll=4)
            pltpu.sync_copy(m_v, m_sp.at[s, pl.ds(ci * _CH, _CH)])
            return prevs

        lax.fori_loop(1, _NCH, _chunk, prevs)

        def _zrow(r, _):
            for k in range(_NSUB):
                out_v[r, pl.ds(k * 16, 16)] = zeros
            return 0

        lax.fori_loop(0, _OCH, _zrow, 0)

        jvec0 = jnp.where(lane == 15, jnp.int32(_N_TEXT - 1), 0)

        def _bchunk(t, carry):
            ci = _NCH - 1 - t
            pltpu.sync_copy(m_sp.at[s, pl.ds(ci * _CH, _CH)], m_v)

            def _bstep(u, carry):
                jj, jvec = carry
                x = ci * _CH + _CH - 1 - u
                row = jnp.full((16,), x - ci * _CH, jnp.int32)
                mv = plsc.load_gather(m_v, [row, jj]).astype(jnp.int32)
                jn = jj - mv
                jvec = jnp.where(lane == ((x - 1) % 16), jn, jvec)

                @pl.when(((x - 1) % 16 == 0) & (x > 0))
                def _store():
                    jcol_v[pl.ds(pl.multiple_of(x - 1, 16), 16)] = jvec

                return jn, jvec

            return lax.fori_loop(0, _CH, _bstep, carry, unroll=8)

        jj0 = jnp.full((16,), _N_TEXT - 1, jnp.int32)
        lax.fori_loop(0, _NCH, _bchunk, (jj0, jvec0))

        def _ochunk(co, _):
            for q in range(_OCH // 16):
                rows = lane + q * 16
                cols = jcol_v[pl.ds(pl.multiple_of(co * _OCH + q * 16, 16), 16)]
                plsc.store_scatter(out_v, [rows, cols], ones)
            pltpu.sync_copy(out_v, out_hbm.at[b, pl.ds(co * _OCH, _OCH)])
            for q in range(_OCH // 16):
                rows = lane + q * 16
                cols = jcol_v[pl.ds(pl.multiple_of(co * _OCH + q * 16, 16), 16)]
                plsc.store_scatter(out_v, [rows, cols], zeros)
            return 0

        lax.fori_loop(0, _N_MEL // _OCH, _ochunk, 0)


@functools.partial(
    pl.kernel,
    out_type=jax.ShapeDtypeStruct((_B, _N_MEL, _N_TEXT), jnp.float32),
    mesh=plsc.VectorSubcoreMesh(core_axis_name="c", subcore_axis_name="s",
                                num_cores=2, num_subcores=16),
    compiler_params=pltpu.CompilerParams(needs_layout_passes=False),
    scratch_types=[
        pltpu.VMEM((2, _CH, _N_TEXT), jnp.float32),
        pltpu.VMEM((_CH, _N_TEXT), jnp.float32),
        pltpu.VMEM((_N_MEL,), jnp.int32),
        pltpu.VMEM((_OCH, _N_TEXT), jnp.float32),
        pltpu.VMEM_SHARED((4, _N_MEL, _N_TEXT), jnp.float32),
        pltpu.SemaphoreType.DMA((2,)),
    ],
)
def _sc_dp(lp_hbm, out_hbm, lp_v, m_v, jcol_v, out_v, m_sp, sems):
    _sc_dp_body(lp_hbm, out_hbm, lp_v, m_v, jcol_v, out_v, m_sp, sems)


def kernel(text_emb, mel_emb):
    b, n_mel, d = mel_emb.shape
    n_text = text_emb.shape[1]
    lp = pl.pallas_call(
        _bmm_kernel,
        grid=(b,),
        in_specs=[
            pl.BlockSpec((1, n_mel, d), lambda i: (i, 0, 0)),
            pl.BlockSpec((1, n_text, d), lambda i: (i, 0, 0)),
        ],
        out_specs=pl.BlockSpec((1, n_mel, n_text), lambda i: (i, 0, 0)),
        out_shape=jax.ShapeDtypeStruct((b, n_mel, n_text), jnp.float32),
    )(mel_emb, text_emb)
    return _sc_dp(lp)
